# baseline (speedup 1.0000x reference)
import jax
import jax.numpy as jnp
from jax.experimental import pallas as pl
from jax.experimental.pallas import tpu as pltpu

B, C, H, W = 32, 384, 14, 14
K = 9
N = H * W
OUT = 2 * C


def _body(x_ref, Wc1_ref, bc1_ref, Wc2_ref, bc2_ref, Wfc_ref, bfc_ref,
          Wio_ref, bio_ref, Wu_ref, bu_ref, Wk0_ref, bk0_ref, Wk1_ref,
          bk1_ref, Wmu_ref, bmu_ref, Wdec_ref, bdec_ref, out_ref,
          cio_ref, cfc_ref, bcomb_ref, w1d_ref, w2d_ref):
    f32 = jnp.float32

    @pl.when(pl.program_id(0) == 0)
    def _precompute():
        wu_top = Wu_ref[0:OUT, :]
        wu_bot = Wu_ref[OUT:2 * OUT, :]
        cio_ref[...] = jnp.dot(Wio_ref[...], wu_top,
                               preferred_element_type=f32).astype(jnp.bfloat16)
        cfc_ref[...] = jnp.dot(Wfc_ref[...], wu_bot,
                               preferred_element_type=f32).astype(jnp.bfloat16)
        bcomb_ref[...] = (jnp.dot(bio_ref[...], wu_top, preferred_element_type=f32)
                          + jnp.dot(bfc_ref[...], wu_bot, preferred_element_type=f32)
                          + bu_ref[...])
        w1d_ref[...] = (Wc1_ref[0:C, :] - Wc1_ref[C:2 * C, :]).astype(jnp.bfloat16)
        w2d_ref[...] = (Wc2_ref[0:C, :] - Wc2_ref[C:2 * C, :]).astype(jnp.bfloat16)

    pts = x_ref[0]

    h = jnp.dot(pts, Wk0_ref[...], preferred_element_type=f32) + bk0_ref[...]
    h = jax.nn.relu(jnp.dot(h, Wk1_ref[...], preferred_element_type=f32) + bk1_ref[...])
    mu = jnp.dot(h, Wmu_ref[...], preferred_element_type=f32) + bmu_ref[...]
    logits = jnp.dot(mu, Wdec_ref[...], preferred_element_type=f32) + bdec_ref[...]
    kio = jax.lax.broadcasted_iota(jnp.int32, logits.shape, 1)
    lmax = jnp.max(logits, axis=1, keepdims=True)
    n_i = jnp.min(jnp.where(logits == lmax, kio, K), axis=1, keepdims=True)

    sq = jnp.sum(pts * pts, axis=1, keepdims=True)
    sq_row = jnp.transpose(sq)
    gram = jax.lax.dot_general(pts, pts, (((1,), (1,)), ((), ())),
                               preferred_element_type=f32)
    dist = (sq + sq_row) - 2.0 * gram

    big = jnp.float32(3.0e38)
    m = jnp.min(dist, axis=1, keepdims=True)
    kth_small = [m]
    for k in range(K - 2):
        m = jnp.min(jnp.where(dist > m, dist, big), axis=1, keepdims=True)
        kth_small.append(m)
    thr = jnp.full_like(kth_small[0], -big)
    for k in range(K - 1):
        thr = jnp.where(n_i == k + 1, kth_small[k], thr)
    a_mat = jnp.where(dist <= thr, 1.0, 0.0)

    deg = n_i.astype(f32)

    bf16 = jnp.bfloat16
    a_bf = a_mat.astype(bf16)
    pts_bf = pts.astype(bf16)

    Wb1 = Wc1_ref[C:2 * C, :].astype(bf16)
    s1 = jnp.dot(a_bf, pts_bf, preferred_element_type=f32)
    h1 = jax.nn.relu(deg * (jnp.dot(pts_bf, w1d_ref[...], preferred_element_type=f32)
                            + bc1_ref[...])
                     + jnp.dot(s1.astype(bf16), Wb1, preferred_element_type=f32))

    Wb2 = Wc2_ref[C:2 * C, :].astype(bf16)
    h1_bf = h1.astype(bf16)
    s2 = jnp.dot(a_bf, h1_bf, preferred_element_type=f32)
    h2 = (deg * (jnp.dot(h1_bf, w2d_ref[...], preferred_element_type=f32) + bc2_ref[...])
          + jnp.dot(s2.astype(bf16), Wb2, preferred_element_type=f32))

    upd = jax.nn.relu(jnp.dot(pts_bf, cio_ref[...], preferred_element_type=f32)
                      + jnp.dot(h2.astype(bf16), cfc_ref[...],
                                preferred_element_type=f32)
                      + bcomb_ref[...])
    out_ref[0] = upd


def _full(shape):
    nd = len(shape)
    return pl.BlockSpec(shape, lambda b: (0,) * nd)


@jax.jit
def kernel(x, Wc1, bc1, Wc2, bc2, Wfc, bfc, Wio, bio, Wu, bu,
           Wk0, bk0, Wk1, bk1, Wmu, bmu, Wdec, bdec):
    xf = x.reshape(B, C, N).transpose(0, 2, 1)
    b2 = lambda v: v.reshape(1, -1)
    ws = [Wc1, b2(bc1), Wc2, b2(bc2), Wfc, b2(bfc), Wio, b2(bio), Wu, b2(bu),
          Wk0, b2(bk0), Wk1, b2(bk1), Wmu, b2(bmu), Wdec, b2(bdec)]
    out = pl.pallas_call(
        _body,
        grid=(B,),
        in_specs=[pl.BlockSpec((1, N, C), lambda b: (b, 0, 0))] +
                 [_full(w.shape) for w in ws],
        out_specs=pl.BlockSpec((1, N, OUT), lambda b: (b, 0, 0)),
        out_shape=jax.ShapeDtypeStruct((B, N, OUT), jnp.float32),
        scratch_shapes=[
            pltpu.VMEM((C, OUT), jnp.bfloat16),
            pltpu.VMEM((C, OUT), jnp.bfloat16),
            pltpu.VMEM((1, OUT), jnp.float32),
            pltpu.VMEM((C, C), jnp.bfloat16),
            pltpu.VMEM((C, C), jnp.bfloat16),
        ],
        compiler_params=pltpu.CompilerParams(
            dimension_semantics=("arbitrary",),
            vmem_limit_bytes=100 * 1024 * 1024,
        ),
    )(xf, *ws)
    return out.transpose(0, 2, 1).reshape(B, OUT, H, W)


# speedup vs baseline: 1.0634x; 1.0634x over previous
import jax
import jax.numpy as jnp
from jax.experimental import pallas as pl
from jax.experimental.pallas import tpu as pltpu

B, C, H, W = 32, 384, 14, 14
K = 9
N = H * W
OUT = 2 * C
IPB = 2


def _body(x_ref, Wc1_ref, bc1_ref, Wc2_ref, bc2_ref, Wfc_ref, bfc_ref,
          Wio_ref, bio_ref, Wu_ref, bu_ref, Wk0_ref, bk0_ref, Wk1_ref,
          bk1_ref, Wmu_ref, bmu_ref, Wdec_ref, bdec_ref, out_ref,
          cio_ref, cfc_ref, bcomb_ref, w1d_ref, w2d_ref):
    f32 = jnp.float32

    @pl.when(pl.program_id(0) == 0)
    def _precompute():
        wu_top = Wu_ref[0:OUT, :]
        wu_bot = Wu_ref[OUT:2 * OUT, :]
        cio_ref[...] = jnp.dot(Wio_ref[...], wu_top,
                               preferred_element_type=f32).astype(jnp.bfloat16)
        cfc_ref[...] = jnp.dot(Wfc_ref[...], wu_bot,
                               preferred_element_type=f32).astype(jnp.bfloat16)
        bcomb_ref[...] = (jnp.dot(bio_ref[...], wu_top, preferred_element_type=f32)
                          + jnp.dot(bfc_ref[...], wu_bot, preferred_element_type=f32)
                          + bu_ref[...])
        w1d_ref[...] = Wc1_ref[0:C, :] - Wc1_ref[C:2 * C, :]
        w2d_ref[...] = Wc2_ref[0:C, :] - Wc2_ref[C:2 * C, :]

    for g in range(IPB):
        _one_image(x_ref[g], Wc1_ref, bc1_ref, Wc2_ref, bc2_ref, Wk0_ref,
                   bk0_ref, Wk1_ref, bk1_ref, Wmu_ref, bmu_ref, Wdec_ref,
                   bdec_ref, out_ref, cio_ref, cfc_ref, bcomb_ref, w1d_ref,
                   w2d_ref, g)


def _one_image(pts, Wc1_ref, bc1_ref, Wc2_ref, bc2_ref, Wk0_ref, bk0_ref,
               Wk1_ref, bk1_ref, Wmu_ref, bmu_ref, Wdec_ref, bdec_ref,
               out_ref, cio_ref, cfc_ref, bcomb_ref, w1d_ref, w2d_ref, g):
    f32 = jnp.float32

    h = jnp.dot(pts, Wk0_ref[...], preferred_element_type=f32) + bk0_ref[...]
    h = jax.nn.relu(jnp.dot(h, Wk1_ref[...], preferred_element_type=f32) + bk1_ref[...])
    mu = jnp.dot(h, Wmu_ref[...], preferred_element_type=f32) + bmu_ref[...]
    logits = jnp.dot(mu, Wdec_ref[...], preferred_element_type=f32) + bdec_ref[...]
    kio = jax.lax.broadcasted_iota(jnp.int32, logits.shape, 1)
    lmax = jnp.max(logits, axis=1, keepdims=True)
    n_i = jnp.min(jnp.where(logits == lmax, kio, K), axis=1, keepdims=True)

    sq = jnp.sum(pts * pts, axis=1, keepdims=True)
    sq_row = jnp.transpose(sq)
    gram = jax.lax.dot_general(pts, pts, (((1,), (1,)), ((), ())),
                               preferred_element_type=f32)
    dist = (sq + sq_row) - 2.0 * gram

    big = jnp.float32(3.0e38)
    m = jnp.min(dist, axis=1, keepdims=True)
    kth_small = [m]
    for k in range(K - 2):
        m = jnp.min(jnp.where(dist > m, dist, big), axis=1, keepdims=True)
        kth_small.append(m)
    thr = jnp.full_like(kth_small[0], -big)
    for k in range(K - 1):
        thr = jnp.where(n_i == k + 1, kth_small[k], thr)
    a_mat = jnp.where(dist <= thr, 1.0, 0.0)

    deg = n_i.astype(f32)

    Wb1 = Wc1_ref[C:2 * C, :]
    s1 = jnp.dot(a_mat, pts, preferred_element_type=f32)
    h1 = jax.nn.relu(deg * (jnp.dot(pts, w1d_ref[...], preferred_element_type=f32)
                            + bc1_ref[...])
                     + jnp.dot(s1, Wb1, preferred_element_type=f32))

    Wb2 = Wc2_ref[C:2 * C, :]
    s2 = jnp.dot(a_mat, h1, preferred_element_type=f32)
    h2 = (deg * (jnp.dot(h1, w2d_ref[...], preferred_element_type=f32) + bc2_ref[...])
          + jnp.dot(s2, Wb2, preferred_element_type=f32))

    upd = jax.nn.relu(jnp.dot(pts.astype(jnp.bfloat16), cio_ref[...],
                              preferred_element_type=f32)
                      + jnp.dot(h2.astype(jnp.bfloat16), cfc_ref[...],
                                preferred_element_type=f32)
                      + bcomb_ref[...])
    out_ref[g] = upd


def _full(shape):
    nd = len(shape)
    return pl.BlockSpec(shape, lambda b: (0,) * nd)


@jax.jit
def kernel(x, Wc1, bc1, Wc2, bc2, Wfc, bfc, Wio, bio, Wu, bu,
           Wk0, bk0, Wk1, bk1, Wmu, bmu, Wdec, bdec):
    xf = x.reshape(B, C, N).transpose(0, 2, 1)
    b2 = lambda v: v.reshape(1, -1)
    ws = [Wc1, b2(bc1), Wc2, b2(bc2), Wfc, b2(bfc), Wio, b2(bio), Wu, b2(bu),
          Wk0, b2(bk0), Wk1, b2(bk1), Wmu, b2(bmu), Wdec, b2(bdec)]
    out = pl.pallas_call(
        _body,
        grid=(B // IPB,),
        in_specs=[pl.BlockSpec((IPB, N, C), lambda b: (b, 0, 0))] +
                 [_full(w.shape) for w in ws],
        out_specs=pl.BlockSpec((IPB, N, OUT), lambda b: (b, 0, 0)),
        out_shape=jax.ShapeDtypeStruct((B, N, OUT), jnp.float32),
        scratch_shapes=[
            pltpu.VMEM((C, OUT), jnp.bfloat16),
            pltpu.VMEM((C, OUT), jnp.bfloat16),
            pltpu.VMEM((1, OUT), jnp.float32),
            pltpu.VMEM((C, C), jnp.float32),
            pltpu.VMEM((C, C), jnp.float32),
        ],
        compiler_params=pltpu.CompilerParams(
            dimension_semantics=("arbitrary",),
            vmem_limit_bytes=100 * 1024 * 1024,
        ),
    )(xf, *ws)
    return out.transpose(0, 2, 1).reshape(B, OUT, H, W)


# speedup vs baseline: 1.0909x; 1.0259x over previous
import jax
import jax.numpy as jnp
from jax.experimental import pallas as pl
from jax.experimental.pallas import tpu as pltpu

B, C, H, W = 32, 384, 14, 14
K = 9
N = H * W
OUT = 2 * C
IPB = 4


def _body(x_ref, Wc1_ref, bc1_ref, Wc2_ref, bc2_ref, Wfc_ref, bfc_ref,
          Wio_ref, bio_ref, Wu_ref, bu_ref, Wk0_ref, bk0_ref, Wk1_ref,
          bk1_ref, Wmu_ref, bmu_ref, Wdec_ref, bdec_ref, out_ref,
          cio_ref, cfc_ref, bcomb_ref, w1d_ref, w2d_ref):
    f32 = jnp.float32

    @pl.when(pl.program_id(0) == 0)
    def _precompute():
        wu_top = Wu_ref[0:OUT, :]
        wu_bot = Wu_ref[OUT:2 * OUT, :]
        cio_ref[...] = jnp.dot(Wio_ref[...], wu_top,
                               preferred_element_type=f32).astype(jnp.bfloat16)
        cfc_ref[...] = jnp.dot(Wfc_ref[...], wu_bot,
                               preferred_element_type=f32).astype(jnp.bfloat16)
        bcomb_ref[...] = (jnp.dot(bio_ref[...], wu_top, preferred_element_type=f32)
                          + jnp.dot(bfc_ref[...], wu_bot, preferred_element_type=f32)
                          + bu_ref[...])
        w1d_ref[...] = Wc1_ref[0:C, :] - Wc1_ref[C:2 * C, :]
        w2d_ref[...] = Wc2_ref[0:C, :] - Wc2_ref[C:2 * C, :]

    for g in range(IPB):
        _one_image(x_ref[g], Wc1_ref, bc1_ref, Wc2_ref, bc2_ref, Wk0_ref,
                   bk0_ref, Wk1_ref, bk1_ref, Wmu_ref, bmu_ref, Wdec_ref,
                   bdec_ref, out_ref, cio_ref, cfc_ref, bcomb_ref, w1d_ref,
                   w2d_ref, g)


def _one_image(pts, Wc1_ref, bc1_ref, Wc2_ref, bc2_ref, Wk0_ref, bk0_ref,
               Wk1_ref, bk1_ref, Wmu_ref, bmu_ref, Wdec_ref, bdec_ref,
               out_ref, cio_ref, cfc_ref, bcomb_ref, w1d_ref, w2d_ref, g):
    f32 = jnp.float32

    h = jnp.dot(pts, Wk0_ref[...], preferred_element_type=f32) + bk0_ref[...]
    h = jax.nn.relu(jnp.dot(h, Wk1_ref[...], preferred_element_type=f32) + bk1_ref[...])
    mu = jnp.dot(h, Wmu_ref[...], preferred_element_type=f32) + bmu_ref[...]
    logits = jnp.dot(mu, Wdec_ref[...], preferred_element_type=f32) + bdec_ref[...]
    kio = jax.lax.broadcasted_iota(jnp.int32, logits.shape, 1)
    lmax = jnp.max(logits, axis=1, keepdims=True)
    n_i = jnp.min(jnp.where(logits == lmax, kio, K), axis=1, keepdims=True)

    sq = jnp.sum(pts * pts, axis=1, keepdims=True)
    sq_row = jnp.transpose(sq)
    gram = jax.lax.dot_general(pts, pts, (((1,), (1,)), ((), ())),
                               preferred_element_type=f32)
    dist = (sq + sq_row) - 2.0 * gram

    big = jnp.float32(3.0e38)
    m = jnp.min(dist, axis=1, keepdims=True)
    kth_small = [m]
    for k in range(K - 2):
        m = jnp.min(jnp.where(dist > m, dist, big), axis=1, keepdims=True)
        kth_small.append(m)
    thr = jnp.full_like(kth_small[0], -big)
    for k in range(K - 1):
        thr = jnp.where(n_i == k + 1, kth_small[k], thr)
    a_mat = jnp.where(dist <= thr, 1.0, 0.0)

    deg = n_i.astype(f32)

    Wb1 = Wc1_ref[C:2 * C, :]
    s1 = jnp.dot(a_mat, pts, preferred_element_type=f32)
    h1 = jax.nn.relu(deg * (jnp.dot(pts, w1d_ref[...], preferred_element_type=f32)
                            + bc1_ref[...])
                     + jnp.dot(s1, Wb1, preferred_element_type=f32))

    Wb2 = Wc2_ref[C:2 * C, :]
    s2 = jnp.dot(a_mat, h1, preferred_element_type=f32)
    h2 = (deg * (jnp.dot(h1, w2d_ref[...], preferred_element_type=f32) + bc2_ref[...])
          + jnp.dot(s2, Wb2, preferred_element_type=f32))

    upd = jax.nn.relu(jnp.dot(pts.astype(jnp.bfloat16), cio_ref[...],
                              preferred_element_type=f32)
                      + jnp.dot(h2.astype(jnp.bfloat16), cfc_ref[...],
                                preferred_element_type=f32)
                      + bcomb_ref[...])
    out_ref[g] = upd


def _full(shape):
    nd = len(shape)
    return pl.BlockSpec(shape, lambda b: (0,) * nd)


@jax.jit
def kernel(x, Wc1, bc1, Wc2, bc2, Wfc, bfc, Wio, bio, Wu, bu,
           Wk0, bk0, Wk1, bk1, Wmu, bmu, Wdec, bdec):
    xf = x.reshape(B, C, N).transpose(0, 2, 1)
    b2 = lambda v: v.reshape(1, -1)
    ws = [Wc1, b2(bc1), Wc2, b2(bc2), Wfc, b2(bfc), Wio, b2(bio), Wu, b2(bu),
          Wk0, b2(bk0), Wk1, b2(bk1), Wmu, b2(bmu), Wdec, b2(bdec)]
    out = pl.pallas_call(
        _body,
        grid=(B // IPB,),
        in_specs=[pl.BlockSpec((IPB, N, C), lambda b: (b, 0, 0))] +
                 [_full(w.shape) for w in ws],
        out_specs=pl.BlockSpec((IPB, N, OUT), lambda b: (b, 0, 0)),
        out_shape=jax.ShapeDtypeStruct((B, N, OUT), jnp.float32),
        scratch_shapes=[
            pltpu.VMEM((C, OUT), jnp.bfloat16),
            pltpu.VMEM((C, OUT), jnp.bfloat16),
            pltpu.VMEM((1, OUT), jnp.float32),
            pltpu.VMEM((C, C), jnp.float32),
            pltpu.VMEM((C, C), jnp.float32),
        ],
        compiler_params=pltpu.CompilerParams(
            dimension_semantics=("arbitrary",),
            vmem_limit_bytes=100 * 1024 * 1024,
        ),
    )(xf, *ws)
    return out.transpose(0, 2, 1).reshape(B, OUT, H, W)


# speedup vs baseline: 1.0970x; 1.0055x over previous
import jax
import jax.numpy as jnp
from jax.experimental import pallas as pl
from jax.experimental.pallas import tpu as pltpu

B, C, H, W = 32, 384, 14, 14
K = 9
N = H * W
OUT = 2 * C
IPB = 8


def _body(x_ref, Wc1_ref, bc1_ref, Wc2_ref, bc2_ref, Wfc_ref, bfc_ref,
          Wio_ref, bio_ref, Wu_ref, bu_ref, Wk0_ref, bk0_ref, Wk1_ref,
          bk1_ref, Wmu_ref, bmu_ref, Wdec_ref, bdec_ref, out_ref,
          cio_ref, cfc_ref, bcomb_ref, w1d_ref, w2d_ref):
    f32 = jnp.float32

    @pl.when(pl.program_id(0) == 0)
    def _precompute():
        wu_top = Wu_ref[0:OUT, :]
        wu_bot = Wu_ref[OUT:2 * OUT, :]
        cio_ref[...] = jnp.dot(Wio_ref[...], wu_top,
                               preferred_element_type=f32).astype(jnp.bfloat16)
        cfc_ref[...] = jnp.dot(Wfc_ref[...], wu_bot,
                               preferred_element_type=f32).astype(jnp.bfloat16)
        bcomb_ref[...] = (jnp.dot(bio_ref[...], wu_top, preferred_element_type=f32)
                          + jnp.dot(bfc_ref[...], wu_bot, preferred_element_type=f32)
                          + bu_ref[...])
        w1d_ref[...] = Wc1_ref[0:C, :] - Wc1_ref[C:2 * C, :]
        w2d_ref[...] = Wc2_ref[0:C, :] - Wc2_ref[C:2 * C, :]

    for g in range(IPB):
        _one_image(x_ref[g], Wc1_ref, bc1_ref, Wc2_ref, bc2_ref, Wk0_ref,
                   bk0_ref, Wk1_ref, bk1_ref, Wmu_ref, bmu_ref, Wdec_ref,
                   bdec_ref, out_ref, cio_ref, cfc_ref, bcomb_ref, w1d_ref,
                   w2d_ref, g)


def _one_image(pts, Wc1_ref, bc1_ref, Wc2_ref, bc2_ref, Wk0_ref, bk0_ref,
               Wk1_ref, bk1_ref, Wmu_ref, bmu_ref, Wdec_ref, bdec_ref,
               out_ref, cio_ref, cfc_ref, bcomb_ref, w1d_ref, w2d_ref, g):
    f32 = jnp.float32

    h = jnp.dot(pts, Wk0_ref[...], preferred_element_type=f32) + bk0_ref[...]
    h = jax.nn.relu(jnp.dot(h, Wk1_ref[...], preferred_element_type=f32) + bk1_ref[...])
    mu = jnp.dot(h, Wmu_ref[...], preferred_element_type=f32) + bmu_ref[...]
    logits = jnp.dot(mu, Wdec_ref[...], preferred_element_type=f32) + bdec_ref[...]
    kio = jax.lax.broadcasted_iota(jnp.int32, logits.shape, 1)
    lmax = jnp.max(logits, axis=1, keepdims=True)
    n_i = jnp.min(jnp.where(logits == lmax, kio, K), axis=1, keepdims=True)

    sq = jnp.sum(pts * pts, axis=1, keepdims=True)
    sq_row = jnp.transpose(sq)
    gram = jax.lax.dot_general(pts, pts, (((1,), (1,)), ((), ())),
                               preferred_element_type=f32)
    dist = (sq + sq_row) - 2.0 * gram

    big = jnp.float32(3.0e38)
    m = jnp.min(dist, axis=1, keepdims=True)
    kth_small = [m]
    for k in range(K - 2):
        m = jnp.min(jnp.where(dist > m, dist, big), axis=1, keepdims=True)
        kth_small.append(m)
    thr = jnp.full_like(kth_small[0], -big)
    for k in range(K - 1):
        thr = jnp.where(n_i == k + 1, kth_small[k], thr)
    a_mat = jnp.where(dist <= thr, 1.0, 0.0)

    deg = n_i.astype(f32)

    Wb1 = Wc1_ref[C:2 * C, :]
    s1 = jnp.dot(a_mat, pts, preferred_element_type=f32)
    h1 = jax.nn.relu(deg * (jnp.dot(pts, w1d_ref[...], preferred_element_type=f32)
                            + bc1_ref[...])
                     + jnp.dot(s1, Wb1, preferred_element_type=f32))

    Wb2 = Wc2_ref[C:2 * C, :]
    s2 = jnp.dot(a_mat, h1, preferred_element_type=f32)
    h2 = (deg * (jnp.dot(h1, w2d_ref[...], preferred_element_type=f32) + bc2_ref[...])
          + jnp.dot(s2, Wb2, preferred_element_type=f32))

    upd = jax.nn.relu(jnp.dot(pts.astype(jnp.bfloat16), cio_ref[...],
                              preferred_element_type=f32)
                      + jnp.dot(h2.astype(jnp.bfloat16), cfc_ref[...],
                                preferred_element_type=f32)
                      + bcomb_ref[...])
    out_ref[g] = upd


def _full(shape):
    nd = len(shape)
    return pl.BlockSpec(shape, lambda b: (0,) * nd)


@jax.jit
def kernel(x, Wc1, bc1, Wc2, bc2, Wfc, bfc, Wio, bio, Wu, bu,
           Wk0, bk0, Wk1, bk1, Wmu, bmu, Wdec, bdec):
    xf = x.reshape(B, C, N).transpose(0, 2, 1)
    b2 = lambda v: v.reshape(1, -1)
    ws = [Wc1, b2(bc1), Wc2, b2(bc2), Wfc, b2(bfc), Wio, b2(bio), Wu, b2(bu),
          Wk0, b2(bk0), Wk1, b2(bk1), Wmu, b2(bmu), Wdec, b2(bdec)]
    out = pl.pallas_call(
        _body,
        grid=(B // IPB,),
        in_specs=[pl.BlockSpec((IPB, N, C), lambda b: (b, 0, 0))] +
                 [_full(w.shape) for w in ws],
        out_specs=pl.BlockSpec((IPB, N, OUT), lambda b: (b, 0, 0)),
        out_shape=jax.ShapeDtypeStruct((B, N, OUT), jnp.float32),
        scratch_shapes=[
            pltpu.VMEM((C, OUT), jnp.bfloat16),
            pltpu.VMEM((C, OUT), jnp.bfloat16),
            pltpu.VMEM((1, OUT), jnp.float32),
            pltpu.VMEM((C, C), jnp.float32),
            pltpu.VMEM((C, C), jnp.float32),
        ],
        compiler_params=pltpu.CompilerParams(
            dimension_semantics=("arbitrary",),
            vmem_limit_bytes=100 * 1024 * 1024,
        ),
    )(xf, *ws)
    return out.transpose(0, 2, 1).reshape(B, OUT, H, W)
